# Initial kernel scaffold; baseline (speedup 1.0000x reference)
import jax
import jax.numpy as jnp
from jax.experimental import pallas as pl


def kernel(x, sasa_residue_wise, rotamers, residue_ids, chain_ids, edge_index, covalent_edge_index, covalent_edge_attr, res_res_BB_dihs, noncovalent_edge_index, noncovalent_edge_attr, distmat_top, distmat_3d, batch, res_tables, sasa_W, sasa_b, rot_table, rot_W, rot_b, pos_table, chain_table, cov_tables, bb_table, bb_W, bb_b, nc_tables):
    raise NotImplementedError("write your pallas kernel here")


# speedup vs baseline: 41.1763x; 41.1763x over previous
import functools

import jax
import jax.numpy as jnp
from jax import lax
from jax.experimental import pallas as pl
from jax.experimental.pallas import tpu as pltpu
from jax.experimental.pallas import tpu_sc as plsc

_HID = 128
_F32 = jnp.float32
_I32 = jnp.int32

_SC_CORES = 2
_SC_SUBCORES = 16
_NW = _SC_CORES * _SC_SUBCORES


def _onehot_eq(col, width):
    i = lax.broadcasted_iota(_I32, (col.shape[0], width), 1)
    return (col == i).astype(_F32)


def _onehot_bucket(vals):
    b = vals.shape[0]
    i = lax.broadcasted_iota(_I32, (b, 74), 1).astype(_F32)
    hi = -180.0 + 5.0 * i
    lo = hi - 5.0
    return ((hi >= vals) & (lo < vals)).astype(_F32)


def _h_body(x_ref, sasa_ref, rot_ref, chain_ref,
            res0_ref, res1_ref, res2_ref, res3_ref,
            sasaw_ref, sasab_ref, rot_tab_ref, rotw_ref, rotb_ref,
            chain_tab_ref, out_ref):
    acc = sasab_ref[...] + rotb_ref[...]
    res_refs = (res0_ref, res1_ref, res2_ref, res3_ref)
    for t in range(4):
        oh = _onehot_eq(x_ref[:, t:t + 1], res_refs[t].shape[0])
        acc = acc + jnp.dot(oh, res_refs[t][...], preferred_element_type=_F32)
    sv = sasa_ref[...]
    sv = jnp.where(sv != sv, 0.0, sv)
    acc = acc + jnp.dot(sv, sasaw_ref[...], preferred_element_type=_F32)
    for k in range(4):
        rk = jnp.dot(rot_tab_ref[...], rotw_ref[k * _HID:(k + 1) * _HID, :],
                     preferred_element_type=_F32)
        oh = _onehot_bucket(rot_ref[:, k:k + 1])
        acc = acc + jnp.dot(oh, rk, preferred_element_type=_F32)
    oh = _onehot_eq(chain_ref[:, 0:1], chain_tab_ref.shape[0])
    acc = acc + jnp.dot(oh, chain_tab_ref[...], preferred_element_type=_F32)
    out_ref[...] = acc


def _cov_body(attr_ref, dih_ref, cov0_ref, cov1_ref,
              bb_tab_ref, bbw_ref, bbb_ref,
              nc0_ref, nc1_ref, nc2_ref, cov_out_ref, nctab_out_ref):
    acc = bbb_ref[...] + jnp.dot(_onehot_eq(attr_ref[:, 0:1], cov0_ref.shape[0]),
                                 cov0_ref[...], preferred_element_type=_F32)
    acc = acc + jnp.dot(_onehot_eq(attr_ref[:, 1:2], cov1_ref.shape[0]),
                        cov1_ref[...], preferred_element_type=_F32)
    for k in range(2):
        bk = jnp.dot(bb_tab_ref[...], bbw_ref[k * _HID:(k + 1) * _HID, :],
                     preferred_element_type=_F32)
        acc = acc + jnp.dot(_onehot_bucket(dih_ref[:, k:k + 1]), bk,
                            preferred_element_type=_F32)
    cov_out_ref[...] = acc
    c = lax.broadcasted_iota(_I32, (64, 1), 0)
    tab = jnp.dot(_onehot_eq(c // 16, nc0_ref.shape[0]), nc0_ref[...],
                  preferred_element_type=_F32)
    tab = tab + jnp.dot(_onehot_eq((c // 4) % 4, nc1_ref.shape[0]), nc1_ref[...],
                        preferred_element_type=_F32)
    tab = tab + jnp.dot(_onehot_eq(c % 4, nc2_ref.shape[0]), nc2_ref[...],
                        preferred_element_type=_F32)
    nctab_out_ref[...] = tab


def _make_sc_kernel(n, en, interpret=False):
    hw = 25
    nb = en // _NW
    k = 400
    n_chunks = nb // k
    mesh = plsc.VectorSubcoreMesh(core_axis_name="c", subcore_axis_name="s")

    @functools.partial(
        pl.kernel,
        out_type=(jax.ShapeDtypeStruct((n, _HID), _F32),
                  jax.ShapeDtypeStruct((en, _HID), _F32)),
        mesh=mesh,
        scratch_types=[
            pltpu.VMEM((k, _HID), _F32),
            pltpu.VMEM((k, _HID), _F32),
            pltpu.VMEM((k,), _I32),
            pltpu.VMEM((k,), _I32),
            pltpu.VMEM((k,), _I32),
            pltpu.VMEM((k,), _I32),
            pltpu.VMEM((k,), _I32),
            pltpu.VMEM((k,), _I32),
            pltpu.VMEM((k,), _I32),
            pltpu.VMEM((k,), _I32),
            pltpu.VMEM((k,), _I32),
            pltpu.VMEM_SHARED((64, _HID), _F32),
            pltpu.SemaphoreType.DMA,
            pltpu.SemaphoreType.DMA,
            pltpu.SemaphoreType.DMA,
            pltpu.SemaphoreType.DMA,
            pltpu.SemaphoreType.DMA,
            pltpu.SemaphoreType.DMA,
            pltpu.SemaphoreType.DMA,
        ],
        interpret=interpret,
    )
    def sck(resid, pos_tab, a0h, a1h, a2h, nctab,
            pos_out, nc_out, bufa, bufb, ridx, idxa, idxb,
            a0a, a1a, a2a, a0b, a1b, a2b, stab,
            sga, sgb, soa, sob, saa, sab, stt):
        wid = lax.axis_index("s") * _SC_CORES + lax.axis_index("c")
        @pl.when(lax.axis_index("s") == 0)
        def _():
            pltpu.async_copy(nctab, stab, stt)

        @pl.when(wid < hw)
        def _():
            base = wid * k
            pltpu.sync_copy(resid.at[pl.ds(base, k)], ridx)
            pltpu.async_copy(pos_tab.at[ridx], bufa, sga).wait()
            pltpu.sync_copy(bufa, pos_out.at[pl.ds(base, k)])

        @pl.when(lax.axis_index("s") == 0)
        def _():
            pltpu.make_async_copy(nctab, stab, stt).wait()

        plsc.subcore_barrier()

        nbase = wid * nb
        slots = ((bufa, idxa, (a0a, a1a, a2a), sga, soa, saa),
                 (bufb, idxb, (a0b, a1b, a2b), sgb, sob, sab))

        def attr_start(i):
            _, _, att, _, _, sa = slots[i % 2]
            off = nbase + i * k
            return (pltpu.async_copy(a0h.at[pl.ds(off, k)], att[0], sa),
                    pltpu.async_copy(a1h.at[pl.ds(off, k)], att[1], sa),
                    pltpu.async_copy(a2h.at[pl.ds(off, k)], att[2], sa))

        def idx_compute(i, handles):
            _, idx, att, _, _, _ = slots[i % 2]
            for h in handles:
                h.wait()

            def mk(j, c2):
                sl = pl.ds(j * 16, 16)
                idx[sl] = att[0][sl] * 16 + att[1][sl] * 4 + att[2][sl]
                return c2

            lax.fori_loop(0, k // 16, mk, 0)

        def gather_start(i):
            buf, idx, _, sg, _, _ = slots[i % 2]
            return pltpu.async_copy(stab.at[idx], buf, sg)

        def out_start(i):
            buf, _, _, _, so, _ = slots[i % 2]
            off = nbase + i * k
            return pltpu.async_copy(buf, nc_out.at[pl.ds(off, k)], so)

        ah = {0: attr_start(0), 1: attr_start(1)}
        idx_compute(0, ah.pop(0))
        gh = {0: gather_start(0)}
        oh = {}
        for i in range(n_chunks):
            gh.pop(i).wait()
            oh[i] = out_start(i)
            if i + 1 < n_chunks:
                idx_compute(i + 1, ah.pop(i + 1))
                if i >= 1:
                    oh.pop(i - 1).wait()
                gh[i + 1] = gather_start(i + 1)
                if i + 2 < n_chunks:
                    ah[i + 2] = attr_start(i + 2)
        oh.pop(n_chunks - 2).wait()
        oh.pop(n_chunks - 1).wait()

    return sck


def _add_body(a_ref, b_ref, out_ref):
    out_ref[...] = a_ref[...] + b_ref[...]


def _h_add(a, b, n, interpret=False):
    blk = 2000
    return pl.pallas_call(
        _add_body,
        grid=(n // blk,),
        in_specs=[pl.BlockSpec((blk, _HID), lambda i: (i, 0)),
                  pl.BlockSpec((blk, _HID), lambda i: (i, 0))],
        out_specs=pl.BlockSpec((blk, _HID), lambda i: (i, 0)),
        out_shape=jax.ShapeDtypeStruct((n, _HID), _F32),
        interpret=interpret,
    )(a, b)


def _h_part(xp, sasap, rotp, chainp, res_tables_p, sasaw_p, sasab_p,
            rot_table, rot_w, rotb2, chain_table, n_pad, interpret=False):
    blk = 2000
    grid = (n_pad // blk,)
    full = lambda shape: pl.BlockSpec(shape, lambda i: (0, 0))
    return pl.pallas_call(
        _h_body,
        grid=grid,
        in_specs=[
            pl.BlockSpec((blk, 4), lambda i: (i, 0)),
            pl.BlockSpec((blk, 4), lambda i: (i, 0)),
            pl.BlockSpec((blk, 4), lambda i: (i, 0)),
            pl.BlockSpec((blk, 1), lambda i: (i, 0)),
            full(res_tables_p[0].shape), full(res_tables_p[1].shape),
            full(res_tables_p[2].shape), full(res_tables_p[3].shape),
            full(sasaw_p.shape), full(sasab_p.shape),
            full(rot_table.shape), full(rot_w.shape), full(rotb2.shape),
            full(chain_table.shape),
        ],
        out_specs=pl.BlockSpec((blk, _HID), lambda i: (i, 0)),
        out_shape=jax.ShapeDtypeStruct((n_pad, _HID), _F32),
        interpret=interpret,
    )(xp, sasap, rotp, chainp, *res_tables_p, sasaw_p, sasab_p,
      rot_table, rot_w, rotb2, chain_table)


def _cov_part(attr, dihs, cov0, cov1, bb_table, bb_w, bbb2,
              nc0, nc1, nc2, ec, interpret=False):
    blk = 2000
    grid = (ec // blk,)
    full = lambda shape: pl.BlockSpec(shape, lambda i: (0, 0))
    return pl.pallas_call(
        _cov_body,
        grid=grid,
        in_specs=[
            pl.BlockSpec((blk, 2), lambda i: (i, 0)),
            pl.BlockSpec((blk, 2), lambda i: (i, 0)),
            full(cov0.shape), full(cov1.shape),
            full(bb_table.shape), full(bb_w.shape), full(bbb2.shape),
            full(nc0.shape), full(nc1.shape), full(nc2.shape),
        ],
        out_specs=(pl.BlockSpec((blk, _HID), lambda i: (i, 0)),
                   pl.BlockSpec((64, _HID), lambda i: (0, 0))),
        out_shape=(jax.ShapeDtypeStruct((ec, _HID), _F32),
                   jax.ShapeDtypeStruct((64, _HID), _F32)),
        interpret=interpret,
    )(attr, dihs, cov0, cov1, bb_table, bb_w, bbb2, nc0, nc1, nc2)


def kernel(x, sasa_residue_wise, rotamers, residue_ids, chain_ids, edge_index,
           covalent_edge_index, covalent_edge_attr, res_res_BB_dihs,
           noncovalent_edge_index, noncovalent_edge_attr, distmat_top,
           distmat_3d, batch, res_tables, sasa_W, sasa_b, rot_table, rot_W,
           rot_b, pos_table, chain_table, cov_tables, bb_table, bb_W, bb_b,
           nc_tables):
    n = x.shape[0]
    ec = covalent_edge_attr.shape[0]
    en = noncovalent_edge_attr.shape[0]

    res_tables_p = tuple(jnp.pad(t, ((0, 0), (0, _HID - t.shape[1])))
                         for t in res_tables)
    sasaw_p = jnp.pad(sasa_W, ((0, 0), (_HID - sasa_W.shape[1], 0)))
    sasab_p = jnp.pad(sasa_b.reshape(1, -1), ((0, 0), (_HID - sasa_b.shape[0], 0)))
    rotb2 = rot_b.reshape(1, _HID)
    bbb2 = bb_b.reshape(1, _HID)
    chain2d = chain_ids.astype(_I32).reshape(n, 1)
    resid = residue_ids.astype(_I32)
    attr_i = covalent_edge_attr.astype(_I32)
    nc_i = noncovalent_edge_attr.astype(_I32)
    a0, a1, a2 = nc_i[:, 0], nc_i[:, 1], nc_i[:, 2]

    cov, nctab = _cov_part(attr_i, res_res_BB_dihs, cov_tables[0],
                           cov_tables[1], bb_table, bb_W, bbb2,
                           nc_tables[0], nc_tables[1], nc_tables[2], ec)
    hpart = _h_part(x.astype(_I32), sasa_residue_wise, rotamers, chain2d,
                    res_tables_p, sasaw_p, sasab_p,
                    rot_table, rot_W, rotb2, chain_table, n)

    sck = _make_sc_kernel(n, en)
    posrows, nc = sck(resid, pos_table, a0, a1, a2, nctab)
    h = _h_add(hpart, posrows, n)
    return h, cov, nc


# speedup vs baseline: 45.2843x; 1.0998x over previous
import functools

import jax
import jax.numpy as jnp
from jax import lax
from jax.experimental import pallas as pl
from jax.experimental.pallas import tpu as pltpu
from jax.experimental.pallas import tpu_sc as plsc

_HID = 128
_F32 = jnp.float32
_I32 = jnp.int32

_SC_CORES = 2
_SC_SUBCORES = 16
_NW = _SC_CORES * _SC_SUBCORES


def _onehot_eq(col, width):
    i = lax.broadcasted_iota(_I32, (col.shape[0], width), 1)
    return (col == i).astype(_F32)


def _onehot_bucket(vals):
    b = vals.shape[0]
    i = lax.broadcasted_iota(_I32, (b, 74), 1).astype(_F32)
    hi = -180.0 + 5.0 * i
    lo = hi - 5.0
    return ((hi >= vals) & (lo < vals)).astype(_F32)


def _h_body(x_ref, sasa_ref, rot_ref, chain_ref,
            res0_ref, res1_ref, res2_ref, res3_ref,
            sasaw_ref, sasab_ref, rot_tab_ref, rotw_ref, rotb_ref,
            chain_tab_ref, out_ref):
    acc = sasab_ref[...] + rotb_ref[...]
    res_refs = (res0_ref, res1_ref, res2_ref, res3_ref)
    for t in range(4):
        oh = _onehot_eq(x_ref[:, t:t + 1], res_refs[t].shape[0])
        acc = acc + jnp.dot(oh, res_refs[t][...], preferred_element_type=_F32)
    sv = sasa_ref[...]
    sv = jnp.where(sv != sv, 0.0, sv)
    acc = acc + jnp.dot(sv, sasaw_ref[...], preferred_element_type=_F32)
    for k in range(4):
        rk = jnp.dot(rot_tab_ref[...], rotw_ref[k * _HID:(k + 1) * _HID, :],
                     preferred_element_type=_F32)
        oh = _onehot_bucket(rot_ref[:, k:k + 1])
        acc = acc + jnp.dot(oh, rk, preferred_element_type=_F32)
    oh = _onehot_eq(chain_ref[:, 0:1], chain_tab_ref.shape[0])
    acc = acc + jnp.dot(oh, chain_tab_ref[...], preferred_element_type=_F32)
    out_ref[...] = acc


def _cov_body(attr_ref, dih_ref, cov0_ref, cov1_ref,
              bb_tab_ref, bbw_ref, bbb_ref, cov_out_ref):
    acc = bbb_ref[...] + jnp.dot(_onehot_eq(attr_ref[:, 0:1], cov0_ref.shape[0]),
                                 cov0_ref[...], preferred_element_type=_F32)
    acc = acc + jnp.dot(_onehot_eq(attr_ref[:, 1:2], cov1_ref.shape[0]),
                        cov1_ref[...], preferred_element_type=_F32)
    for k in range(2):
        bk = jnp.dot(bb_tab_ref[...], bbw_ref[k * _HID:(k + 1) * _HID, :],
                     preferred_element_type=_F32)
        acc = acc + jnp.dot(_onehot_bucket(dih_ref[:, k:k + 1]), bk,
                            preferred_element_type=_F32)
    cov_out_ref[...] = acc


def _make_sc_kernel(n, en, interpret=False):
    hw = 25
    nb = en // _NW
    k = 400
    n_chunks = nb // k
    mesh = plsc.VectorSubcoreMesh(core_axis_name="c", subcore_axis_name="s")

    @functools.partial(
        pl.kernel,
        out_type=(jax.ShapeDtypeStruct((n, _HID), _F32),
                  jax.ShapeDtypeStruct((en, _HID), _F32)),
        mesh=mesh,
        scratch_types=[
            pltpu.VMEM((k, _HID), _F32),
            pltpu.VMEM((k, _HID), _F32),
            pltpu.VMEM((k,), _I32),
            pltpu.VMEM((k,), _I32),
            pltpu.VMEM((k,), _I32),
            pltpu.VMEM((k,), _I32),
            pltpu.VMEM((k,), _I32),
            pltpu.VMEM((k,), _I32),
            pltpu.VMEM((k,), _I32),
            pltpu.VMEM((k,), _I32),
            pltpu.VMEM((k,), _I32),
            pltpu.VMEM((1, _HID), _F32),
            pltpu.VMEM((1, _HID), _F32),
            pltpu.VMEM((4, _HID), _F32),
            pltpu.VMEM((4, _HID), _F32),
            pltpu.VMEM_SHARED((64, _HID), _F32),
            pltpu.SemaphoreType.DMA,
            pltpu.SemaphoreType.DMA,
            pltpu.SemaphoreType.DMA,
            pltpu.SemaphoreType.DMA,
            pltpu.SemaphoreType.DMA,
            pltpu.SemaphoreType.DMA,
        ],
        interpret=interpret,
    )
    def sck(resid, pos_tab, a0h, a1h, a2h, nc0h, nc1h, nc2h,
            pos_out, nc_out, bufa, bufb, ridx, idxa, idxb,
            a0a, a1a, a2a, a0b, a1b, a2b, t0v, t1v, t2v, tbv, stab,
            sga, sgb, soa, sob, saa, sab):
        s = lax.axis_index("s")
        wid = s * _SC_CORES + lax.axis_index("c")
        pltpu.sync_copy(nc0h.at[pl.ds(s // 4, 1)], t0v)
        pltpu.sync_copy(nc1h.at[pl.ds(s % 4, 1)], t1v)
        pltpu.sync_copy(nc2h.at[pl.ds(0, 4)], t2v)
        for q in range(4):
            for j in range(_HID // 16):
                sl = pl.ds(j * 16, 16)
                tbv[q, sl] = t0v[0, sl] + t1v[0, sl] + t2v[q, sl]
        pltpu.sync_copy(tbv, stab.at[pl.ds(s * 4, 4)])

        @pl.when(wid < hw)
        def _():
            base = wid * k
            pltpu.sync_copy(resid.at[pl.ds(base, k)], ridx)
            pltpu.async_copy(pos_tab.at[ridx], bufa, sga).wait()
            pltpu.sync_copy(bufa, pos_out.at[pl.ds(base, k)])

        plsc.subcore_barrier()

        nbase = wid * nb
        slots = ((bufa, idxa, (a0a, a1a, a2a), sga, soa, saa),
                 (bufb, idxb, (a0b, a1b, a2b), sgb, sob, sab))

        def attr_start(i):
            _, _, att, _, _, sa = slots[i % 2]
            off = nbase + i * k
            return (pltpu.async_copy(a0h.at[pl.ds(off, k)], att[0], sa),
                    pltpu.async_copy(a1h.at[pl.ds(off, k)], att[1], sa),
                    pltpu.async_copy(a2h.at[pl.ds(off, k)], att[2], sa))

        def idx_compute(i, handles):
            _, idx, att, _, _, _ = slots[i % 2]
            for h in handles:
                h.wait()

            def mk(j, c2):
                sl = pl.ds(j * 16, 16)
                idx[sl] = att[0][sl] * 16 + att[1][sl] * 4 + att[2][sl]
                return c2

            lax.fori_loop(0, k // 16, mk, 0)

        def gather_start(i):
            buf, idx, _, sg, _, _ = slots[i % 2]
            return pltpu.async_copy(stab.at[idx], buf, sg)

        def out_start(i):
            buf, _, _, _, so, _ = slots[i % 2]
            off = nbase + i * k
            return pltpu.async_copy(buf, nc_out.at[pl.ds(off, k)], so)

        ah = {0: attr_start(0), 1: attr_start(1)}
        idx_compute(0, ah.pop(0))
        gh = {0: gather_start(0)}
        oh = {}
        for i in range(n_chunks):
            gh.pop(i).wait()
            oh[i] = out_start(i)
            if i + 1 < n_chunks:
                idx_compute(i + 1, ah.pop(i + 1))
                if i >= 1:
                    oh.pop(i - 1).wait()
                gh[i + 1] = gather_start(i + 1)
                if i + 2 < n_chunks:
                    ah[i + 2] = attr_start(i + 2)
        oh.pop(n_chunks - 2).wait()
        oh.pop(n_chunks - 1).wait()

    return sck


def _add_body(a_ref, b_ref, out_ref):
    out_ref[...] = a_ref[...] + b_ref[...]


def _h_add(a, b, n, interpret=False):
    blk = 2000
    return pl.pallas_call(
        _add_body,
        grid=(n // blk,),
        in_specs=[pl.BlockSpec((blk, _HID), lambda i: (i, 0)),
                  pl.BlockSpec((blk, _HID), lambda i: (i, 0))],
        out_specs=pl.BlockSpec((blk, _HID), lambda i: (i, 0)),
        out_shape=jax.ShapeDtypeStruct((n, _HID), _F32),
        interpret=interpret,
    )(a, b)


def _h_part(xp, sasap, rotp, chainp, res_tables_p, sasaw_p, sasab_p,
            rot_table, rot_w, rotb2, chain_table, n_pad, interpret=False):
    blk = 2000
    grid = (n_pad // blk,)
    full = lambda shape: pl.BlockSpec(shape, lambda i: (0, 0))
    return pl.pallas_call(
        _h_body,
        grid=grid,
        in_specs=[
            pl.BlockSpec((blk, 4), lambda i: (i, 0)),
            pl.BlockSpec((blk, 4), lambda i: (i, 0)),
            pl.BlockSpec((blk, 4), lambda i: (i, 0)),
            pl.BlockSpec((blk, 1), lambda i: (i, 0)),
            full(res_tables_p[0].shape), full(res_tables_p[1].shape),
            full(res_tables_p[2].shape), full(res_tables_p[3].shape),
            full(sasaw_p.shape), full(sasab_p.shape),
            full(rot_table.shape), full(rot_w.shape), full(rotb2.shape),
            full(chain_table.shape),
        ],
        out_specs=pl.BlockSpec((blk, _HID), lambda i: (i, 0)),
        out_shape=jax.ShapeDtypeStruct((n_pad, _HID), _F32),
        interpret=interpret,
    )(xp, sasap, rotp, chainp, *res_tables_p, sasaw_p, sasab_p,
      rot_table, rot_w, rotb2, chain_table)


def _cov_part(attr, dihs, cov0, cov1, bb_table, bb_w, bbb2, ec,
              interpret=False):
    blk = 2000
    grid = (ec // blk,)
    full = lambda shape: pl.BlockSpec(shape, lambda i: (0, 0))
    return pl.pallas_call(
        _cov_body,
        grid=grid,
        in_specs=[
            pl.BlockSpec((blk, 2), lambda i: (i, 0)),
            pl.BlockSpec((blk, 2), lambda i: (i, 0)),
            full(cov0.shape), full(cov1.shape),
            full(bb_table.shape), full(bb_w.shape), full(bbb2.shape),
        ],
        out_specs=pl.BlockSpec((blk, _HID), lambda i: (i, 0)),
        out_shape=jax.ShapeDtypeStruct((ec, _HID), _F32),
        interpret=interpret,
    )(attr, dihs, cov0, cov1, bb_table, bb_w, bbb2)


def kernel(x, sasa_residue_wise, rotamers, residue_ids, chain_ids, edge_index,
           covalent_edge_index, covalent_edge_attr, res_res_BB_dihs,
           noncovalent_edge_index, noncovalent_edge_attr, distmat_top,
           distmat_3d, batch, res_tables, sasa_W, sasa_b, rot_table, rot_W,
           rot_b, pos_table, chain_table, cov_tables, bb_table, bb_W, bb_b,
           nc_tables):
    n = x.shape[0]
    ec = covalent_edge_attr.shape[0]
    en = noncovalent_edge_attr.shape[0]

    res_tables_p = tuple(jnp.pad(t, ((0, 0), (0, _HID - t.shape[1])))
                         for t in res_tables)
    sasaw_p = jnp.pad(sasa_W, ((0, 0), (_HID - sasa_W.shape[1], 0)))
    sasab_p = jnp.pad(sasa_b.reshape(1, -1), ((0, 0), (_HID - sasa_b.shape[0], 0)))
    rotb2 = rot_b.reshape(1, _HID)
    bbb2 = bb_b.reshape(1, _HID)
    chain2d = chain_ids.astype(_I32).reshape(n, 1)
    resid = residue_ids.astype(_I32)
    attr_i = covalent_edge_attr.astype(_I32)
    nc_i = noncovalent_edge_attr.astype(_I32)
    a0, a1, a2 = nc_i[:, 0], nc_i[:, 1], nc_i[:, 2]

    cov = _cov_part(attr_i, res_res_BB_dihs, cov_tables[0],
                    cov_tables[1], bb_table, bb_W, bbb2, ec)
    hpart = _h_part(x.astype(_I32), sasa_residue_wise, rotamers, chain2d,
                    res_tables_p, sasaw_p, sasab_p,
                    rot_table, rot_W, rotb2, chain_table, n)

    sck = _make_sc_kernel(n, en)
    posrows, nc = sck(resid, pos_table, a0, a1, a2,
                      nc_tables[0], nc_tables[1], nc_tables[2])
    h = _h_add(hpart, posrows, n)
    return h, cov, nc


# speedup vs baseline: 45.4442x; 1.0035x over previous
import functools

import jax
import jax.numpy as jnp
from jax import lax
from jax.experimental import pallas as pl
from jax.experimental.pallas import tpu as pltpu
from jax.experimental.pallas import tpu_sc as plsc

_HID = 128
_F32 = jnp.float32
_I32 = jnp.int32

_SC_CORES = 2
_SC_SUBCORES = 16
_NW = _SC_CORES * _SC_SUBCORES


def _onehot_eq(col, width):
    i = lax.broadcasted_iota(_I32, (col.shape[0], width), 1)
    return (col == i).astype(_F32)


def _onehot_bucket(vals):
    b = vals.shape[0]
    i = lax.broadcasted_iota(_I32, (b, 74), 1).astype(_F32)
    hi = -180.0 + 5.0 * i
    lo = hi - 5.0
    return ((hi >= vals) & (lo < vals)).astype(_F32)


def _h_body(x_ref, sasa_ref, rot_ref, chain_ref,
            res0_ref, res1_ref, res2_ref, res3_ref,
            sasaw_ref, sasab_ref, rot_tab_ref, rotw_ref, rotb_ref,
            chain_tab_ref, out_ref):
    acc = sasab_ref[...] + rotb_ref[...]
    res_refs = (res0_ref, res1_ref, res2_ref, res3_ref)
    for t in range(4):
        oh = _onehot_eq(x_ref[:, t:t + 1], res_refs[t].shape[0])
        acc = acc + jnp.dot(oh, res_refs[t][...], preferred_element_type=_F32)
    sv = sasa_ref[...]
    sv = jnp.where(sv != sv, 0.0, sv)
    acc = acc + jnp.dot(sv, sasaw_ref[...], preferred_element_type=_F32)
    for k in range(4):
        rk = jnp.dot(rot_tab_ref[...], rotw_ref[k * _HID:(k + 1) * _HID, :],
                     preferred_element_type=_F32)
        oh = _onehot_bucket(rot_ref[:, k:k + 1])
        acc = acc + jnp.dot(oh, rk, preferred_element_type=_F32)
    oh = _onehot_eq(chain_ref[:, 0:1], chain_tab_ref.shape[0])
    acc = acc + jnp.dot(oh, chain_tab_ref[...], preferred_element_type=_F32)
    out_ref[...] = acc


def _cov_body(attr_ref, dih_ref, cov0_ref, cov1_ref,
              bb_tab_ref, bbw_ref, bbb_ref, cov_out_ref):
    acc = bbb_ref[...] + jnp.dot(_onehot_eq(attr_ref[:, 0:1], cov0_ref.shape[0]),
                                 cov0_ref[...], preferred_element_type=_F32)
    acc = acc + jnp.dot(_onehot_eq(attr_ref[:, 1:2], cov1_ref.shape[0]),
                        cov1_ref[...], preferred_element_type=_F32)
    for k in range(2):
        bk = jnp.dot(bb_tab_ref[...], bbw_ref[k * _HID:(k + 1) * _HID, :],
                     preferred_element_type=_F32)
        acc = acc + jnp.dot(_onehot_bucket(dih_ref[:, k:k + 1]), bk,
                            preferred_element_type=_F32)
    cov_out_ref[...] = acc


def _make_sc_kernel(n, en, interpret=False):
    k = 400
    hw = n // k
    nb = en // _NW
    n_chunks = nb // k
    mesh = plsc.VectorSubcoreMesh(core_axis_name="c", subcore_axis_name="s")

    @functools.partial(
        pl.kernel,
        out_type=(jax.ShapeDtypeStruct((n, _HID), _F32),
                  jax.ShapeDtypeStruct((en, _HID), _F32)),
        mesh=mesh,
        scratch_types=[
            pltpu.VMEM((k, _HID), _F32),
            pltpu.VMEM((k, _HID), _F32),
            pltpu.VMEM((k,), _I32),
            pltpu.VMEM((k,), _I32),
            pltpu.VMEM((k,), _I32),
            pltpu.VMEM((k,), _I32),
            pltpu.VMEM((k,), _I32),
            pltpu.VMEM((k,), _I32),
            pltpu.VMEM((k,), _I32),
            pltpu.VMEM((k,), _I32),
            pltpu.VMEM((k,), _I32),
            pltpu.VMEM((1, _HID), _F32),
            pltpu.VMEM((1, _HID), _F32),
            pltpu.VMEM((4, _HID), _F32),
            pltpu.VMEM((4, _HID), _F32),
            pltpu.VMEM_SHARED((64, _HID), _F32),
            pltpu.SemaphoreType.DMA,
            pltpu.SemaphoreType.DMA,
            pltpu.SemaphoreType.DMA,
            pltpu.SemaphoreType.DMA,
            pltpu.SemaphoreType.DMA,
            pltpu.SemaphoreType.DMA,
        ],
        interpret=interpret,
    )
    def sck(resid, pos_tab, a0h, a1h, a2h, nc0h, nc1h, nc2h,
            pos_out, nc_out, bufa, bufb, ridx, idxa, idxb,
            a0a, a1a, a2a, a0b, a1b, a2b, t0v, t1v, t2v, tbv, stab,
            sga, sgb, soa, sob, saa, sab):
        s = lax.axis_index("s")
        wid = s * _SC_CORES + lax.axis_index("c")
        pltpu.sync_copy(nc0h.at[pl.ds(s // 4, 1)], t0v)
        pltpu.sync_copy(nc1h.at[pl.ds(s % 4, 1)], t1v)
        pltpu.sync_copy(nc2h.at[pl.ds(0, 4)], t2v)
        for q in range(4):
            for j in range(_HID // 16):
                sl = pl.ds(j * 16, 16)
                tbv[q, sl] = t0v[0, sl] + t1v[0, sl] + t2v[q, sl]
        pltpu.sync_copy(tbv, stab.at[pl.ds(s * 4, 4)])

        @pl.when(wid < hw)
        def _():
            base = wid * k
            pltpu.sync_copy(resid.at[pl.ds(base, k)], ridx)
            pltpu.async_copy(pos_tab.at[ridx], bufa, sga).wait()
            pltpu.sync_copy(bufa, pos_out.at[pl.ds(base, k)])

        plsc.subcore_barrier()

        nbase = wid * nb
        slots = ((bufa, idxa, (a0a, a1a, a2a), sga, soa, saa),
                 (bufb, idxb, (a0b, a1b, a2b), sgb, sob, sab))

        def attr_start(i):
            _, _, att, _, _, sa = slots[i % 2]
            off = nbase + i * k
            return (pltpu.async_copy(a0h.at[pl.ds(off, k)], att[0], sa),
                    pltpu.async_copy(a1h.at[pl.ds(off, k)], att[1], sa),
                    pltpu.async_copy(a2h.at[pl.ds(off, k)], att[2], sa))

        def idx_compute(i, handles):
            _, idx, att, _, _, _ = slots[i % 2]
            for h in handles:
                h.wait()

            def mk(j, c2):
                sl = pl.ds(j * 16, 16)
                idx[sl] = att[0][sl] * 16 + att[1][sl] * 4 + att[2][sl]
                return c2

            lax.fori_loop(0, k // 16, mk, 0)

        def gather_start(i):
            buf, idx, _, sg, _, _ = slots[i % 2]
            return pltpu.async_copy(stab.at[idx], buf, sg)

        def out_start(i):
            buf, _, _, _, so, _ = slots[i % 2]
            off = nbase + i * k
            return pltpu.async_copy(buf, nc_out.at[pl.ds(off, k)], so)

        ah = {0: attr_start(0), 1: attr_start(1)}
        idx_compute(0, ah.pop(0))
        gh = {0: gather_start(0)}
        oh = {}
        for i in range(n_chunks):
            gh.pop(i).wait()
            oh[i] = out_start(i)
            if i + 1 < n_chunks:
                idx_compute(i + 1, ah.pop(i + 1))
                if i >= 1:
                    oh.pop(i - 1).wait()
                gh[i + 1] = gather_start(i + 1)
                if i + 2 < n_chunks:
                    ah[i + 2] = attr_start(i + 2)
        oh.pop(n_chunks - 2).wait()
        oh.pop(n_chunks - 1).wait()

    return sck


def _add_body(a_ref, b_ref, out_ref):
    out_ref[...] = a_ref[...] + b_ref[...]


def _h_add(a, b, n, interpret=False):
    blk = 2000
    return pl.pallas_call(
        _add_body,
        grid=(n // blk,),
        in_specs=[pl.BlockSpec((blk, _HID), lambda i: (i, 0)),
                  pl.BlockSpec((blk, _HID), lambda i: (i, 0))],
        out_specs=pl.BlockSpec((blk, _HID), lambda i: (i, 0)),
        out_shape=jax.ShapeDtypeStruct((n, _HID), _F32),
        interpret=interpret,
    )(a, b)


def _h_part(xp, sasap, rotp, chainp, res_tables_p, sasaw_p, sasab_p,
            rot_table, rot_w, rotb2, chain_table, n_pad, interpret=False):
    blk = 2000
    grid = (n_pad // blk,)
    full = lambda shape: pl.BlockSpec(shape, lambda i: (0, 0))
    return pl.pallas_call(
        _h_body,
        grid=grid,
        in_specs=[
            pl.BlockSpec((blk, 4), lambda i: (i, 0)),
            pl.BlockSpec((blk, 4), lambda i: (i, 0)),
            pl.BlockSpec((blk, 4), lambda i: (i, 0)),
            pl.BlockSpec((blk, 1), lambda i: (i, 0)),
            full(res_tables_p[0].shape), full(res_tables_p[1].shape),
            full(res_tables_p[2].shape), full(res_tables_p[3].shape),
            full(sasaw_p.shape), full(sasab_p.shape),
            full(rot_table.shape), full(rot_w.shape), full(rotb2.shape),
            full(chain_table.shape),
        ],
        out_specs=pl.BlockSpec((blk, _HID), lambda i: (i, 0)),
        out_shape=jax.ShapeDtypeStruct((n_pad, _HID), _F32),
        interpret=interpret,
    )(xp, sasap, rotp, chainp, *res_tables_p, sasaw_p, sasab_p,
      rot_table, rot_w, rotb2, chain_table)


def _cov_part(attr, dihs, cov0, cov1, bb_table, bb_w, bbb2, ec,
              interpret=False):
    blk = 2000
    grid = (ec // blk,)
    full = lambda shape: pl.BlockSpec(shape, lambda i: (0, 0))
    return pl.pallas_call(
        _cov_body,
        grid=grid,
        in_specs=[
            pl.BlockSpec((blk, 2), lambda i: (i, 0)),
            pl.BlockSpec((blk, 2), lambda i: (i, 0)),
            full(cov0.shape), full(cov1.shape),
            full(bb_table.shape), full(bb_w.shape), full(bbb2.shape),
        ],
        out_specs=pl.BlockSpec((blk, _HID), lambda i: (i, 0)),
        out_shape=jax.ShapeDtypeStruct((ec, _HID), _F32),
        interpret=interpret,
    )(attr, dihs, cov0, cov1, bb_table, bb_w, bbb2)


def kernel(x, sasa_residue_wise, rotamers, residue_ids, chain_ids, edge_index,
           covalent_edge_index, covalent_edge_attr, res_res_BB_dihs,
           noncovalent_edge_index, noncovalent_edge_attr, distmat_top,
           distmat_3d, batch, res_tables, sasa_W, sasa_b, rot_table, rot_W,
           rot_b, pos_table, chain_table, cov_tables, bb_table, bb_W, bb_b,
           nc_tables):
    n = x.shape[0]
    ec = covalent_edge_attr.shape[0]
    en = noncovalent_edge_attr.shape[0]

    res_tables_p = tuple(jnp.pad(t, ((0, 0), (0, _HID - t.shape[1])))
                         for t in res_tables)
    sasaw_p = jnp.pad(sasa_W, ((0, 0), (_HID - sasa_W.shape[1], 0)))
    sasab_p = jnp.pad(sasa_b.reshape(1, -1), ((0, 0), (_HID - sasa_b.shape[0], 0)))
    rotb2 = rot_b.reshape(1, _HID)
    bbb2 = bb_b.reshape(1, _HID)
    chain2d = chain_ids.astype(_I32).reshape(n, 1)
    resid = residue_ids.astype(_I32)
    attr_i = covalent_edge_attr.astype(_I32)
    nc_i = noncovalent_edge_attr.astype(_I32)
    a0, a1, a2 = nc_i[:, 0], nc_i[:, 1], nc_i[:, 2]

    cov = _cov_part(attr_i, res_res_BB_dihs, cov_tables[0],
                    cov_tables[1], bb_table, bb_W, bbb2, ec)
    hpart = _h_part(x.astype(_I32), sasa_residue_wise, rotamers, chain2d,
                    res_tables_p, sasaw_p, sasab_p,
                    rot_table, rot_W, rotb2, chain_table, n)

    sck = _make_sc_kernel(n, en)
    posrows, nc = sck(resid, pos_table, a0, a1, a2,
                      nc_tables[0], nc_tables[1], nc_tables[2])
    h = _h_add(hpart, posrows, n)
    return h, cov, nc
